# Initial kernel scaffold; baseline (speedup 1.0000x reference)
import jax
import jax.numpy as jnp
from jax.experimental import pallas as pl


def kernel(inputs, W_base, b_base, W_gate, A, B):
    raise NotImplementedError("write your pallas kernel here")


# speedup vs baseline: 2.6420x; 2.6420x over previous
import functools

import jax
import jax.numpy as jnp
from jax.experimental import pallas as pl
from jax.experimental.pallas import tpu as pltpu

D_MODEL = 4096
OUT_FEATURES = 4096
NUM_EXPERTS = 8
TOP_K = 2
LORA_RANK = 16
SCALING = 32 / 16

TM = 512
TN = 512


def _fused_kernel(x_ref, wb_ref, b_ref, wg_ref, aall_ref, ball_ref,
                  out_ref, hw_ref):
    n = pl.program_id(0)
    m = pl.program_id(1)
    x = x_ref[...]

    @pl.when(n == 0)
    def _router_and_lora_a():
        logits = jax.lax.dot_general(
            x, wg_ref[...], (((1,), (1,)), ((), ())),
            preferred_element_type=jnp.float32)
        mx = jnp.max(logits, axis=1, keepdims=True)
        e = jnp.exp(logits - mx)
        probs = e / jnp.sum(e, axis=1, keepdims=True)
        iota8 = jax.lax.broadcasted_iota(jnp.int32, (TM, NUM_EXPERTS), 1)
        m1 = jnp.max(probs, axis=1, keepdims=True)
        i1 = jnp.min(jnp.where(probs == m1, iota8, NUM_EXPERTS),
                     axis=1, keepdims=True)
        masked = jnp.where(iota8 == i1, -1.0, probs)
        m2 = jnp.max(masked, axis=1, keepdims=True)
        i2 = jnp.min(jnp.where(masked == m2, iota8, NUM_EXPERTS),
                     axis=1, keepdims=True)
        scale = SCALING / (m1 + m2)
        grp = jax.lax.broadcasted_iota(
            jnp.int32, (TM, NUM_EXPERTS * LORA_RANK), 1) // LORA_RANK
        w128 = (jnp.where(grp == i1, m1, 0.0)
                + jnp.where(grp == i2, m2, 0.0)) * scale
        h = jax.lax.dot_general(
            x, aall_ref[...], (((1,), (1,)), ((), ())),
            preferred_element_type=jnp.float32)
        hw_ref[pl.ds(m * TM, TM), :] = h * w128

    acc = jax.lax.dot_general(
        x, wb_ref[...], (((1,), (1,)), ((), ())),
        preferred_element_type=jnp.float32)
    acc += jax.lax.dot_general(
        hw_ref[pl.ds(m * TM, TM), :], ball_ref[...],
        (((1,), (0,)), ((), ())),
        preferred_element_type=jnp.float32)
    out_ref[...] = acc + b_ref[...]


@jax.jit
def kernel(inputs, W_base, b_base, W_gate, A, B):
    lead = inputs.shape[:-1]
    x = inputs.reshape(-1, D_MODEL)
    M = x.shape[0]
    A_all = A.reshape(NUM_EXPERTS * LORA_RANK, D_MODEL)
    B_all = B.transpose(0, 2, 1).reshape(NUM_EXPERTS * LORA_RANK,
                                         OUT_FEATURES)
    b2 = b_base.reshape(1, OUT_FEATURES)

    grid = (OUT_FEATURES // TN, M // TM)
    out = pl.pallas_call(
        _fused_kernel,
        grid=grid,
        in_specs=[
            pl.BlockSpec((TM, D_MODEL), lambda n, m: (m, 0)),
            pl.BlockSpec((TN, D_MODEL), lambda n, m: (n, 0)),
            pl.BlockSpec((1, TN), lambda n, m: (0, n)),
            pl.BlockSpec((NUM_EXPERTS, D_MODEL), lambda n, m: (0, 0)),
            pl.BlockSpec((NUM_EXPERTS * LORA_RANK, D_MODEL),
                         lambda n, m: (0, 0)),
            pl.BlockSpec((NUM_EXPERTS * LORA_RANK, TN),
                         lambda n, m: (0, n)),
        ],
        out_specs=pl.BlockSpec((TM, TN), lambda n, m: (m, n)),
        out_shape=jax.ShapeDtypeStruct((M, OUT_FEATURES), jnp.float32),
        scratch_shapes=[pltpu.VMEM((M, NUM_EXPERTS * LORA_RANK),
                                   jnp.float32)],
        compiler_params=pltpu.CompilerParams(
            dimension_semantics=("arbitrary", "arbitrary"),
        ),
    )(x, W_base, b2, W_gate, A_all, B_all)
    return out.reshape(lead + (OUT_FEATURES,))


# speedup vs baseline: 2.7049x; 1.0238x over previous
import functools

import jax
import jax.numpy as jnp
from jax.experimental import pallas as pl
from jax.experimental.pallas import tpu as pltpu

D_MODEL = 4096
OUT_FEATURES = 4096
NUM_EXPERTS = 8
TOP_K = 2
LORA_RANK = 16
SCALING = 32 / 16

TM = 512
TN = 1024


def _fused_kernel(x_ref, wb_ref, b_ref, wg_ref, aall_ref, ball_ref,
                  out_ref, hw_ref):
    n = pl.program_id(0)
    m = pl.program_id(1)
    x = x_ref[...]

    @pl.when(n == 0)
    def _router_and_lora_a():
        logits = jax.lax.dot_general(
            x, wg_ref[...], (((1,), (1,)), ((), ())),
            preferred_element_type=jnp.float32)
        mx = jnp.max(logits, axis=1, keepdims=True)
        e = jnp.exp(logits - mx)
        probs = e / jnp.sum(e, axis=1, keepdims=True)
        iota8 = jax.lax.broadcasted_iota(jnp.int32, (TM, NUM_EXPERTS), 1)
        m1 = jnp.max(probs, axis=1, keepdims=True)
        i1 = jnp.min(jnp.where(probs == m1, iota8, NUM_EXPERTS),
                     axis=1, keepdims=True)
        masked = jnp.where(iota8 == i1, -1.0, probs)
        m2 = jnp.max(masked, axis=1, keepdims=True)
        i2 = jnp.min(jnp.where(masked == m2, iota8, NUM_EXPERTS),
                     axis=1, keepdims=True)
        scale = SCALING / (m1 + m2)
        grp = jax.lax.broadcasted_iota(
            jnp.int32, (TM, NUM_EXPERTS * LORA_RANK), 1) // LORA_RANK
        w128 = (jnp.where(grp == i1, m1, 0.0)
                + jnp.where(grp == i2, m2, 0.0)) * scale
        h = jax.lax.dot_general(
            x, aall_ref[...], (((1,), (1,)), ((), ())),
            preferred_element_type=jnp.float32)
        hw_ref[pl.ds(m * TM, TM), :] = (h * w128).astype(jnp.bfloat16)

    acc = jax.lax.dot_general(
        x, wb_ref[...], (((1,), (1,)), ((), ())),
        preferred_element_type=jnp.float32)
    acc += jax.lax.dot_general(
        hw_ref[pl.ds(m * TM, TM), :], ball_ref[...],
        (((1,), (0,)), ((), ())),
        preferred_element_type=jnp.float32)
    out_ref[...] = acc + b_ref[...]


@jax.jit
def kernel(inputs, W_base, b_base, W_gate, A, B):
    lead = inputs.shape[:-1]
    x = inputs.reshape(-1, D_MODEL).astype(jnp.bfloat16)
    M = x.shape[0]
    A_all = A.reshape(NUM_EXPERTS * LORA_RANK, D_MODEL).astype(jnp.bfloat16)
    B_all = B.transpose(0, 2, 1).reshape(
        NUM_EXPERTS * LORA_RANK, OUT_FEATURES).astype(jnp.bfloat16)
    W_base = W_base.astype(jnp.bfloat16)
    W_gate = W_gate.astype(jnp.bfloat16)
    b2 = b_base.reshape(1, OUT_FEATURES)

    grid = (OUT_FEATURES // TN, M // TM)
    out = pl.pallas_call(
        _fused_kernel,
        grid=grid,
        in_specs=[
            pl.BlockSpec((TM, D_MODEL), lambda n, m: (m, 0)),
            pl.BlockSpec((TN, D_MODEL), lambda n, m: (n, 0)),
            pl.BlockSpec((1, TN), lambda n, m: (0, n)),
            pl.BlockSpec((NUM_EXPERTS, D_MODEL), lambda n, m: (0, 0)),
            pl.BlockSpec((NUM_EXPERTS * LORA_RANK, D_MODEL),
                         lambda n, m: (0, 0)),
            pl.BlockSpec((NUM_EXPERTS * LORA_RANK, TN),
                         lambda n, m: (0, n)),
        ],
        out_specs=pl.BlockSpec((TM, TN), lambda n, m: (m, n)),
        out_shape=jax.ShapeDtypeStruct((M, OUT_FEATURES), jnp.float32),
        scratch_shapes=[pltpu.VMEM((M, NUM_EXPERTS * LORA_RANK),
                                   jnp.bfloat16)],
        compiler_params=pltpu.CompilerParams(
            dimension_semantics=("arbitrary", "arbitrary"),
        ),
    )(x, W_base, b2, W_gate, A_all, B_all)
    return out.reshape(lead + (OUT_FEATURES,))
